# baseline (speedup 1.0000x reference)
import functools

import jax
import jax.numpy as jnp
from jax import lax
from jax.experimental import pallas as pl
from jax.experimental.pallas import tpu as pltpu
from jax.experimental.pallas import tpu_sc as plsc

VOCAB = 1000000
EMBED_DIM = 32
B = 16384
L = 50

N_IDX = B * L
IDX_MINOR = 128
N_IDX_ROWS = N_IDX // IDX_MINOR

NUM_WORKERS = 32
ROWS_PER_WORKER = N_IDX_ROWS // NUM_WORKERS

CHUNK_IDX_ROWS = 8
CHUNK = CHUNK_IDX_ROWS * IDX_MINOR
CHUNKS_PER_WORKER = ROWS_PER_WORKER // CHUNK_IDX_ROWS


def _sc_gather_relu(idx2d, table):
    mesh = plsc.VectorSubcoreMesh(core_axis_name="c", subcore_axis_name="s")

    @functools.partial(
        pl.kernel,
        mesh=mesh,
        out_type=jax.ShapeDtypeStruct((N_IDX, EMBED_DIM), jnp.float32),
        scratch_types=[
            pltpu.VMEM((ROWS_PER_WORKER, IDX_MINOR), jnp.int32),
            pltpu.VMEM((CHUNK, EMBED_DIM), jnp.float32),
            pltpu.SemaphoreType.DMA,
        ],
        compiler_params=pltpu.CompilerParams(use_tc_tiling_on_sc=False),
    )
    def k(idx_hbm, table_hbm, out_hbm, idx_v, rows_v, sem):
        wid = lax.axis_index("c") * 16 + lax.axis_index("s")
        row_base = wid * ROWS_PER_WORKER

        pltpu.sync_copy(idx_hbm.at[pl.ds(row_base, ROWS_PER_WORKER)], idx_v)

        def chunk_body(g, carry):
            copies = []
            for j in range(CHUNK_IDX_ROWS):
                copies.append(
                    pltpu.async_copy(
                        table_hbm.at[idx_v.at[g * CHUNK_IDX_ROWS + j]],
                        rows_v.at[pl.ds(j * IDX_MINOR, IDX_MINOR)],
                        sem,
                    )
                )
            for c in copies:
                c.wait()

            def relu_row(r, c2):
                a = rows_v[r, pl.ds(0, 16)]
                b = rows_v[r, pl.ds(16, 16)]
                rows_v[r, pl.ds(0, 16)] = jnp.maximum(a, 0.0)
                rows_v[r, pl.ds(16, 16)] = jnp.maximum(b, 0.0)
                return c2

            lax.fori_loop(0, CHUNK, relu_row, 0)

            out_base = row_base * IDX_MINOR + g * CHUNK
            pltpu.sync_copy(rows_v, out_hbm.at[pl.ds(out_base, CHUNK)])
            return carry

        lax.fori_loop(0, CHUNKS_PER_WORKER, chunk_body, 0)

    return k(idx2d, table)


def kernel(predicate_indices, embed_weight):
    idx2d = predicate_indices.astype(jnp.int32).reshape(N_IDX_ROWS, IDX_MINOR)
    flat = _sc_gather_relu(idx2d, embed_weight)
    return flat.reshape(B, L, EMBED_DIM)


# speedup vs baseline: 1.5499x; 1.5499x over previous
import functools

import jax
import jax.numpy as jnp
from jax import lax
from jax.experimental import pallas as pl
from jax.experimental.pallas import tpu as pltpu
from jax.experimental.pallas import tpu_sc as plsc

VOCAB = 1000000
EMBED_DIM = 32
B = 16384
L = 50

NUM_WORKERS = 32
ROWS_PER_WORKER = B // NUM_WORKERS

CHUNK_ROWS = 16
CHUNKS_PER_WORKER = ROWS_PER_WORKER // CHUNK_ROWS


def _sc_gather_relu(idx, table):
    mesh = plsc.VectorSubcoreMesh(core_axis_name="c", subcore_axis_name="s")

    @functools.partial(
        pl.kernel,
        mesh=mesh,
        out_type=jax.ShapeDtypeStruct((B, L, EMBED_DIM), jnp.float32),
        scratch_types=[
            pltpu.VMEM((ROWS_PER_WORKER, L), jnp.int32),
            pltpu.VMEM((CHUNK_ROWS, L, EMBED_DIM), jnp.float32),
            pltpu.SemaphoreType.DMA,
        ],
        compiler_params=pltpu.CompilerParams(use_tc_tiling_on_sc=False),
    )
    def k(idx_hbm, table_hbm, out_hbm, idx_v, rows_v, sem):
        wid = lax.axis_index("c") * 16 + lax.axis_index("s")
        row_base = wid * ROWS_PER_WORKER

        pltpu.sync_copy(idx_hbm.at[pl.ds(row_base, ROWS_PER_WORKER)], idx_v)

        def chunk_body(g, carry):
            copies = []
            for j in range(CHUNK_ROWS):
                copies.append(
                    pltpu.async_copy(
                        table_hbm.at[idx_v.at[g * CHUNK_ROWS + j]],
                        rows_v.at[j],
                        sem,
                    )
                )
            for c in copies:
                c.wait()

            for j in range(CHUNK_ROWS):

                def relu_row(r, c2, j=j):
                    a = rows_v[j, r, pl.ds(0, 16)]
                    b = rows_v[j, r, pl.ds(16, 16)]
                    rows_v[j, r, pl.ds(0, 16)] = jnp.maximum(a, 0.0)
                    rows_v[j, r, pl.ds(16, 16)] = jnp.maximum(b, 0.0)
                    return c2

                lax.fori_loop(0, L, relu_row, 0)

            pltpu.sync_copy(rows_v, out_hbm.at[pl.ds(row_base + g * CHUNK_ROWS, CHUNK_ROWS)])
            return carry

        lax.fori_loop(0, CHUNKS_PER_WORKER, chunk_body, 0)

    return k(idx, table)


def kernel(predicate_indices, embed_weight):
    return _sc_gather_relu(predicate_indices.astype(jnp.int32), embed_weight)


# speedup vs baseline: 1.7300x; 1.1162x over previous
import functools

import jax
import jax.numpy as jnp
from jax import lax
from jax.experimental import pallas as pl
from jax.experimental.pallas import tpu as pltpu
from jax.experimental.pallas import tpu_sc as plsc

VOCAB = 1000000
EMBED_DIM = 32
B = 16384
L = 50

NUM_WORKERS = 32
ROWS_PER_WORKER = B // NUM_WORKERS

CHUNK_ROWS = 8
NCHUNKS = ROWS_PER_WORKER // CHUNK_ROWS


def _sc_gather_relu(idx, table):
    mesh = plsc.VectorSubcoreMesh(core_axis_name="c", subcore_axis_name="s")

    @functools.partial(
        pl.kernel,
        mesh=mesh,
        out_type=jax.ShapeDtypeStruct((B, L, EMBED_DIM), jnp.float32),
        scratch_types=[
            pltpu.VMEM((ROWS_PER_WORKER, L), jnp.int32),
            pltpu.VMEM((CHUNK_ROWS, L, EMBED_DIM), jnp.float32),
            pltpu.VMEM((CHUNK_ROWS, L, EMBED_DIM), jnp.float32),
            pltpu.VMEM((CHUNK_ROWS, L, EMBED_DIM), jnp.float32),
            pltpu.VMEM((CHUNK_ROWS, L, EMBED_DIM), jnp.float32),
            pltpu.SemaphoreType.DMA,
            pltpu.SemaphoreType.DMA,
            pltpu.SemaphoreType.DMA,
            pltpu.SemaphoreType.DMA,
        ],
        compiler_params=pltpu.CompilerParams(use_tc_tiling_on_sc=False),
    )
    def k(idx_hbm, table_hbm, out_hbm, idx_v, g0, g1, o0, o1,
          gs0, gs1, os0, os1):
        wid = lax.axis_index("c") * 16 + lax.axis_index("s")
        row_base = wid * ROWS_PER_WORKER

        gbuf = (g0, g1)
        obuf = (o0, o1)
        gsem = (gs0, gs1)
        osem = (os0, os1)

        pltpu.sync_copy(idx_hbm.at[pl.ds(row_base, ROWS_PER_WORKER)], idx_v)

        def fire_gather(g, b):
            for j in range(CHUNK_ROWS):
                pltpu.make_async_copy(
                    table_hbm.at[idx_v.at[g * CHUNK_ROWS + j]],
                    gbuf[b].at[j],
                    gsem[b],
                ).start()

        def wait_gather(b):
            for j in range(CHUNK_ROWS):
                pltpu.make_async_copy(
                    table_hbm.at[idx_v.at[j]],
                    gbuf[b].at[j],
                    gsem[b],
                ).wait()

        def relu_chunk(b):
            src = gbuf[b]
            dst = obuf[b]

            def body(r, c):
                for j in range(CHUNK_ROWS):
                    a0 = src[j, r, pl.ds(0, 16)]
                    a1 = src[j, r, pl.ds(16, 16)]
                    dst[j, r, pl.ds(0, 16)] = jnp.maximum(a0, 0.0)
                    dst[j, r, pl.ds(16, 16)] = jnp.maximum(a1, 0.0)
                return c

            lax.fori_loop(0, L, body, 0)

        def fire_out(g, b):
            pltpu.make_async_copy(
                obuf[b],
                out_hbm.at[pl.ds(row_base + g * CHUNK_ROWS, CHUNK_ROWS)],
                osem[b],
            ).start()

        def drain_out(b):
            pltpu.make_async_copy(
                obuf[b],
                out_hbm.at[pl.ds(row_base, CHUNK_ROWS)],
                osem[b],
            ).wait()

        fire_gather(0, 0)
        fire_gather(1, 1)

        def loop_body(i, carry):
            g = i * 2
            for b in (0, 1):
                wait_gather(b)

                @pl.when(g + b >= 2)
                def _():
                    drain_out(b)

                relu_chunk(b)
                fire_out(g + b, b)

                @pl.when(g + b + 2 < NCHUNKS)
                def _(b=b):
                    fire_gather(g + b + 2, b)

            return carry

        lax.fori_loop(0, NCHUNKS // 2, loop_body, 0)

        drain_out(0)
        drain_out(1)

    return k(idx, table)


def kernel(predicate_indices, embed_weight):
    return _sc_gather_relu(predicate_indices.astype(jnp.int32), embed_weight)
